# baseline (speedup 1.0000x reference)
import functools

import jax
import jax.numpy as jnp
from jax import lax
from jax.experimental import pallas as pl
from jax.experimental.pallas import tpu as pltpu
from jax.experimental.pallas import tpu_sc as plsc

_B = 16384
_F = 26
_V = 100000
_D = 32
_N = _B * _F
_NC = 2
_NS = 16
_NW = _NC * _NS
_RPW = _N // _NW
_CHUNK = 1664
_NCHUNK = _RPW // _CHUNK
_G = 128
_GPC = _CHUNK // _G
_PAT = 208

_mesh = plsc.VectorSubcoreMesh(
    core_axis_name="c", subcore_axis_name="s",
    num_cores=_NC, num_subcores=_NS)


@functools.partial(
    pl.kernel,
    out_type=jax.ShapeDtypeStruct((_N, _D), jnp.float32),
    mesh=_mesh,
    scratch_types=[
        pltpu.VMEM((_CHUNK,), jnp.int32),
        pltpu.VMEM((_CHUNK, _D), jnp.float32),
        pltpu.VMEM((_PAT,), jnp.int32),
        pltpu.SemaphoreType.DMA,
    ],
    compiler_params=pltpu.CompilerParams(use_tc_tiling_on_sc=False),
)
def _gather_kernel(tab_hbm, idx_hbm, out_hbm, idx_v, rows_v, pat_v, sem):
    wid = lax.axis_index("s") * _NC + lax.axis_index("c")
    lane = lax.iota(jnp.int32, 16)
    for g in range(_PAT // 16):
        pat_v[pl.ds(g * 16, 16)] = ((lane + g * 16) % _F) * _V
    base = wid * _RPW

    def chunk_body(c, carry):
        row0 = base + c * _CHUNK
        pltpu.sync_copy(idx_hbm.at[pl.ds(row0, _CHUNK)], idx_v)
        for g in range(_CHUNK // 16):
            slot = (g % (_PAT // 16)) * 16
            idx_v[pl.ds(g * 16, 16)] = (
                idx_v[pl.ds(g * 16, 16)] + pat_v[pl.ds(slot, 16)])
        copies = []
        for j in range(_GPC):
            cp = pltpu.make_async_copy(
                tab_hbm.at[idx_v.at[pl.ds(j * _G, _G)]],
                rows_v.at[pl.ds(j * _G, _G)], sem)
            cp.start()
            copies.append(cp)
        for cp in copies:
            cp.wait()
        pltpu.sync_copy(rows_v, out_hbm.at[pl.ds(row0, _CHUNK)])
        return carry

    lax.fori_loop(0, _NCHUNK, chunk_body, None)


def kernel(indices, tables):
    idx2 = indices.reshape(_N)
    tab = tables.reshape(_F * _V, _D)
    out = _gather_kernel(tab, idx2)
    return out.reshape(_B, _F * _D)
